# Initial kernel scaffold; baseline (speedup 1.0000x reference)
import functools

import jax
import jax.numpy as jnp
from jax import lax
from jax.experimental import pallas as pl
from jax.experimental.pallas import tpu as pltpu
from jax.experimental.pallas import tpu_sc as plsc

N = 10000
E = 320000
D_IN = 128
D_H = 128
D_OUT = 64

N1 = 10240
RPT = N1 // 16
NW = 32
NB = 79
B = 128
EPW = NB * B
EPAD = NW * EPW

_mesh = plsc.VectorSubcoreMesh(core_axis_name="c", subcore_axis_name="s")


def _wid():
    return lax.axis_index("c") * 16 + lax.axis_index("s")


@functools.partial(
    pl.kernel,
    out_type=(
        jax.ShapeDtypeStruct((2, N1, 16), jnp.float32),
        jax.ShapeDtypeStruct((2, N1, 16), jnp.float32),
    ),
    mesh=_mesh,
    scratch_types=[
        pltpu.VMEM((NB, B), jnp.int32),
        pltpu.VMEM((NB, B), jnp.int32),
        pltpu.VMEM((B, 16), jnp.float32),
        pltpu.VMEM_SHARED((N1, 16), jnp.float32),
        pltpu.VMEM_SHARED((N1, 16), jnp.float32),
    ],
)
def _deg_kernel(src_hbm, dst_hbm, ones_hbm, z16_hbm,
                deg_out_hbm, deg_in_hbm,
                src_v, dst_v, ones_v, acc_o, acc_i):
    c = lax.axis_index("c")
    s = lax.axis_index("s")
    w = _wid()
    r0 = s * RPT
    pltpu.sync_copy(z16_hbm, acc_o.at[pl.ds(r0, RPT)])
    pltpu.sync_copy(z16_hbm, acc_i.at[pl.ds(r0, RPT)])
    pltpu.sync_copy(src_hbm.at[w], src_v)
    pltpu.sync_copy(dst_hbm.at[w], dst_v)
    pltpu.sync_copy(ones_hbm, ones_v)
    plsc.subcore_barrier()

    def body(j, carry):
        pltpu.sync_copy(ones_v, acc_o.at[src_v.at[j]], add=True)
        pltpu.sync_copy(ones_v, acc_i.at[dst_v.at[j]], add=True)
        return carry

    lax.fori_loop(0, NB, body, 0)
    plsc.subcore_barrier()
    pltpu.sync_copy(acc_o.at[pl.ds(r0, RPT)], deg_out_hbm.at[c, pl.ds(r0, RPT)])
    pltpu.sync_copy(acc_i.at[pl.ds(r0, RPT)], deg_in_hbm.at[c, pl.ds(r0, RPT)])


def _make_agg(D):
    @functools.partial(
        pl.kernel,
        out_type=jax.ShapeDtypeStruct((2, N1, D), jnp.float32),
        mesh=_mesh,
        scratch_types=[
            pltpu.VMEM((NB, B), jnp.int32),
            pltpu.VMEM((NB, B), jnp.int32),
            pltpu.VMEM((B, D), jnp.float32),
            pltpu.VMEM((B, D), jnp.float32),
            pltpu.VMEM_SHARED((N1, D), jnp.float32),
            pltpu.SemaphoreType.DMA,
            pltpu.SemaphoreType.DMA,
        ],
    )
    def agg_kernel(m_hbm, src_hbm, dst_hbm, z_hbm, out_hbm,
                   src_v, dst_v, rows0, rows1, acc, sem0, sem1):
        c = lax.axis_index("c")
        s = lax.axis_index("s")
        w = _wid()
        r0 = s * RPT
        pltpu.sync_copy(z_hbm, acc.at[pl.ds(r0, RPT)])
        pltpu.sync_copy(src_hbm.at[w], src_v)
        pltpu.sync_copy(dst_hbm.at[w], dst_v)
        plsc.subcore_barrier()

        pltpu.async_copy(m_hbm.at[src_v.at[0]], rows0, sem0)

        def body(p, carry):
            j0 = 2 * p

            @pl.when(p < NB // 2)
            def _fire1():
                pltpu.async_copy(m_hbm.at[src_v.at[j0 + 1]], rows1, sem1)

            pltpu.make_async_copy(m_hbm.at[src_v.at[0]], rows0, sem0).wait()
            pltpu.sync_copy(rows0, acc.at[dst_v.at[j0]], add=True)

            @pl.when(p < NB // 2)
            def _odd():
                pltpu.async_copy(m_hbm.at[src_v.at[j0 + 2]], rows0, sem0)
                pltpu.make_async_copy(m_hbm.at[src_v.at[0]], rows1, sem1).wait()
                pltpu.sync_copy(rows1, acc.at[dst_v.at[j0 + 1]], add=True)

            return carry

        lax.fori_loop(0, (NB + 1) // 2, body, 0)
        plsc.subcore_barrier()
        pltpu.sync_copy(acc.at[pl.ds(r0, RPT)], out_hbm.at[c, pl.ds(r0, RPT)])

    return agg_kernel


_agg128 = _make_agg(D_H)
_agg64 = _make_agg(D_OUT)


def _norm_col(deg_ref):
    d = deg_ref[0] + deg_ref[1]
    return lax.rsqrt(jnp.maximum(d[:, 0:1], 1.0))


def _mm1_body(dego_ref, x_ref, w_ref, h_ref):
    ns = _norm_col(dego_ref)
    h = jnp.dot(x_ref[...], w_ref[...], preferred_element_type=jnp.float32)
    h_ref[...] = h * ns


def _mm2_body(aggp_ref, dego_ref, degi_ref, w_ref, b_ref, h_ref):
    ns = _norm_col(dego_ref)
    nd = _norm_col(degi_ref)
    t = (aggp_ref[0] + aggp_ref[1]) * nd + b_ref[...]
    t = jnp.maximum(t, 0.0)
    h_ref[...] = jnp.dot(t, w_ref[...], preferred_element_type=jnp.float32) * ns


def _fin_body(aggp_ref, degi_ref, b_ref, o_ref):
    nd = _norm_col(degi_ref)
    o_ref[...] = (aggp_ref[0] + aggp_ref[1]) * nd + b_ref[...]


_BLK = 640
_GRID = N1 // _BLK


def _deg_spec():
    return pl.BlockSpec((2, _BLK, 16), lambda i: (0, i, 0))


def _tc_mm1(deg_out_p, x_pad, W1):
    return pl.pallas_call(
        _mm1_body,
        grid=(_GRID,),
        in_specs=[
            _deg_spec(),
            pl.BlockSpec((_BLK, D_IN), lambda i: (i, 0)),
            pl.BlockSpec((D_IN, D_H), lambda i: (0, 0)),
        ],
        out_specs=pl.BlockSpec((_BLK, D_H), lambda i: (i, 0)),
        out_shape=jax.ShapeDtypeStruct((N1, D_H), jnp.float32),
    )(deg_out_p, x_pad, W1)


def _tc_mm2(agg1_p, deg_out_p, deg_in_p, W2, b1):
    return pl.pallas_call(
        _mm2_body,
        grid=(_GRID,),
        in_specs=[
            pl.BlockSpec((2, _BLK, D_H), lambda i: (0, i, 0)),
            _deg_spec(),
            _deg_spec(),
            pl.BlockSpec((D_H, D_OUT), lambda i: (0, 0)),
            pl.BlockSpec((1, D_H), lambda i: (0, 0)),
        ],
        out_specs=pl.BlockSpec((_BLK, D_OUT), lambda i: (i, 0)),
        out_shape=jax.ShapeDtypeStruct((N1, D_OUT), jnp.float32),
    )(agg1_p, deg_out_p, deg_in_p, W2, b1)


def _tc_fin(agg2_p, deg_in_p, b2):
    return pl.pallas_call(
        _fin_body,
        grid=(_GRID,),
        in_specs=[
            pl.BlockSpec((2, _BLK, D_OUT), lambda i: (0, i, 0)),
            _deg_spec(),
            pl.BlockSpec((1, D_OUT), lambda i: (0, 0)),
        ],
        out_specs=pl.BlockSpec((_BLK, D_OUT), lambda i: (i, 0)),
        out_shape=jax.ShapeDtypeStruct((N1, D_OUT), jnp.float32),
    )(agg2_p, deg_in_p, b2)


def kernel(x, edge_index, W1, b1, W2, b2):
    src = edge_index[0]
    dst = edge_index[1]
    pad = jnp.full((EPAD - E,), N, dtype=jnp.int32)
    src_p = jnp.concatenate([src, pad]).reshape(NW, NB, B)
    dst_p = jnp.concatenate([dst, pad]).reshape(NW, NB, B)
    x_pad = jnp.zeros((N1, D_IN), jnp.float32).at[:N].set(x)

    ones16 = jnp.ones((B, 16), jnp.float32)
    z16 = jnp.zeros((RPT, 16), jnp.float32)
    z128 = jnp.zeros((RPT, D_H), jnp.float32)
    z64 = jnp.zeros((RPT, D_OUT), jnp.float32)

    deg_out_p, deg_in_p = _deg_kernel(src_p, dst_p, ones16, z16)
    h1 = _tc_mm1(deg_out_p, x_pad, W1)
    agg1_p = _agg128(h1, src_p, dst_p, z128)
    h2 = _tc_mm2(agg1_p, deg_out_p, deg_in_p, W2, b1.reshape(1, D_H))
    agg2_p = _agg64(h2, src_p, dst_p, z64)
    out = _tc_fin(agg2_p, deg_in_p, b2.reshape(1, D_OUT))
    return out[:N]


# speedup vs baseline: 5.9326x; 5.9326x over previous
import functools

import jax
import jax.numpy as jnp
from jax import lax
from jax.experimental import pallas as pl
from jax.experimental.pallas import tpu as pltpu
from jax.experimental.pallas import tpu_sc as plsc

N = 10000
E = 320000
D_IN = 128
D_H = 128
D_OUT = 64

N1 = 10240
RPT = N1 // 16
NW = 32
NB = 79
B = 128
EPW = NB * B
EPAD = NW * EPW

_mesh = plsc.VectorSubcoreMesh(core_axis_name="c", subcore_axis_name="s")


def _wid():
    return lax.axis_index("c") * 16 + lax.axis_index("s")


@functools.partial(
    pl.kernel,
    out_type=(
        jax.ShapeDtypeStruct((2, N1, 16), jnp.float32),
        jax.ShapeDtypeStruct((2, N1, 16), jnp.float32),
    ),
    mesh=_mesh,
    compiler_params=pltpu.CompilerParams(use_tc_tiling_on_sc=False),
    scratch_types=[
        pltpu.VMEM((NB, B), jnp.int32),
        pltpu.VMEM((NB, B), jnp.int32),
        pltpu.VMEM((B, 16), jnp.float32),
        pltpu.VMEM_SHARED((N1, 16), jnp.float32),
        pltpu.VMEM_SHARED((N1, 16), jnp.float32),
    ],
)
def _deg_kernel(src_hbm, dst_hbm, ones_hbm, z16_hbm,
                deg_out_hbm, deg_in_hbm,
                src_v, dst_v, ones_v, acc_o, acc_i):
    c = lax.axis_index("c")
    s = lax.axis_index("s")
    w = _wid()
    r0 = s * RPT
    pltpu.sync_copy(z16_hbm, acc_o.at[pl.ds(r0, RPT)])
    pltpu.sync_copy(z16_hbm, acc_i.at[pl.ds(r0, RPT)])
    pltpu.sync_copy(src_hbm.at[w], src_v)
    pltpu.sync_copy(dst_hbm.at[w], dst_v)
    pltpu.sync_copy(ones_hbm, ones_v)
    plsc.subcore_barrier()

    def body(j, carry):
        pltpu.sync_copy(ones_v, acc_o.at[src_v.at[j]], add=True)
        pltpu.sync_copy(ones_v, acc_i.at[dst_v.at[j]], add=True)
        return carry

    lax.fori_loop(0, NB, body, 0)
    plsc.subcore_barrier()
    pltpu.sync_copy(acc_o.at[pl.ds(r0, RPT)], deg_out_hbm.at[c, pl.ds(r0, RPT)])
    pltpu.sync_copy(acc_i.at[pl.ds(r0, RPT)], deg_in_hbm.at[c, pl.ds(r0, RPT)])


def _make_agg(D):
    params = None
    if D != 128:
        params = pltpu.CompilerParams(use_tc_tiling_on_sc=False)

    @functools.partial(
        pl.kernel,
        out_type=jax.ShapeDtypeStruct((2, N1, D), jnp.float32),
        mesh=_mesh,
        compiler_params=params,
        scratch_types=[
            pltpu.VMEM((NB, B), jnp.int32),
            pltpu.VMEM((NB, B), jnp.int32),
            pltpu.VMEM((B, D), jnp.float32),
            pltpu.VMEM_SHARED((N1, D), jnp.float32),
        ],
    )
    def agg_kernel(m_hbm, src_hbm, dst_hbm, z_hbm, out_hbm,
                   src_v, dst_v, rows0, acc):
        c = lax.axis_index("c")
        s = lax.axis_index("s")
        w = _wid()
        r0 = s * RPT
        pltpu.sync_copy(z_hbm, acc.at[pl.ds(r0, RPT)])
        pltpu.sync_copy(src_hbm.at[w], src_v)
        pltpu.sync_copy(dst_hbm.at[w], dst_v)
        plsc.subcore_barrier()

        def body(j, carry):
            pltpu.sync_copy(m_hbm.at[src_v.at[j]], rows0)
            pltpu.sync_copy(rows0, acc.at[dst_v.at[j]], add=True)
            return carry

        lax.fori_loop(0, NB, body, 0)
        plsc.subcore_barrier()
        pltpu.sync_copy(acc.at[pl.ds(r0, RPT)], out_hbm.at[c, pl.ds(r0, RPT)])

    return agg_kernel


_agg128 = _make_agg(D_H)
_agg64 = _make_agg(D_OUT)


def _norm_col(deg_ref):
    d = deg_ref[0] + deg_ref[1]
    return lax.rsqrt(jnp.maximum(d[:, 0:1], 1.0))


def _mm1_body(dego_ref, x_ref, w_ref, h_ref):
    ns = _norm_col(dego_ref)
    h = jnp.dot(x_ref[...], w_ref[...], preferred_element_type=jnp.float32,
                precision=lax.Precision.HIGHEST)
    h_ref[...] = h * ns


def _mm2_body(aggp_ref, dego_ref, degi_ref, w_ref, b_ref, h_ref):
    ns = _norm_col(dego_ref)
    nd = _norm_col(degi_ref)
    t = (aggp_ref[0] + aggp_ref[1]) * nd + b_ref[...]
    t = jnp.maximum(t, 0.0)
    h_ref[...] = jnp.dot(t, w_ref[...], preferred_element_type=jnp.float32,
                            precision=lax.Precision.HIGHEST) * ns


def _fin_body(aggp_ref, degi_ref, b_ref, o_ref):
    nd = _norm_col(degi_ref)
    o_ref[...] = (aggp_ref[0] + aggp_ref[1]) * nd + b_ref[...]


_BLK = 640
_GRID = N1 // _BLK


def _deg_spec():
    return pl.BlockSpec((2, _BLK, 16), lambda i: (0, i, 0))


def _tc_mm1(deg_out_p, x_pad, W1):
    return pl.pallas_call(
        _mm1_body,
        grid=(_GRID,),
        in_specs=[
            _deg_spec(),
            pl.BlockSpec((_BLK, D_IN), lambda i: (i, 0)),
            pl.BlockSpec((D_IN, D_H), lambda i: (0, 0)),
        ],
        out_specs=pl.BlockSpec((_BLK, D_H), lambda i: (i, 0)),
        out_shape=jax.ShapeDtypeStruct((N1, D_H), jnp.float32),
    )(deg_out_p, x_pad, W1)


def _tc_mm2(agg1_p, deg_out_p, deg_in_p, W2, b1):
    return pl.pallas_call(
        _mm2_body,
        grid=(_GRID,),
        in_specs=[
            pl.BlockSpec((2, _BLK, D_H), lambda i: (0, i, 0)),
            _deg_spec(),
            _deg_spec(),
            pl.BlockSpec((D_H, D_OUT), lambda i: (0, 0)),
            pl.BlockSpec((1, D_H), lambda i: (0, 0)),
        ],
        out_specs=pl.BlockSpec((_BLK, D_OUT), lambda i: (i, 0)),
        out_shape=jax.ShapeDtypeStruct((N1, D_OUT), jnp.float32),
    )(agg1_p, deg_out_p, deg_in_p, W2, b1)


def _tc_fin(agg2_p, deg_in_p, b2):
    return pl.pallas_call(
        _fin_body,
        grid=(_GRID,),
        in_specs=[
            pl.BlockSpec((2, _BLK, D_OUT), lambda i: (0, i, 0)),
            _deg_spec(),
            pl.BlockSpec((1, D_OUT), lambda i: (0, 0)),
        ],
        out_specs=pl.BlockSpec((_BLK, D_OUT), lambda i: (i, 0)),
        out_shape=jax.ShapeDtypeStruct((N1, D_OUT), jnp.float32),
    )(agg2_p, deg_in_p, b2)


def kernel(x, edge_index, W1, b1, W2, b2):
    src = edge_index[0]
    dst = edge_index[1]
    pad = jnp.full((EPAD - E,), N, dtype=jnp.int32)
    src_p = jnp.concatenate([src, pad]).reshape(NW, NB, B)
    dst_p = jnp.concatenate([dst, pad]).reshape(NW, NB, B)
    x_pad = jnp.zeros((N1, D_IN), jnp.float32).at[:N].set(x)

    ones16 = jnp.ones((B, 16), jnp.float32)
    z16 = jnp.zeros((RPT, 16), jnp.float32)
    z128 = jnp.zeros((RPT, D_H), jnp.float32)
    z64 = jnp.zeros((RPT, D_OUT), jnp.float32)

    deg_out_p, deg_in_p = _deg_kernel(src_p, dst_p, ones16, z16)
    h1 = _tc_mm1(deg_out_p, x_pad, W1)
    agg1_p = _agg128(h1, src_p, dst_p, z128)
    h2 = _tc_mm2(agg1_p, deg_out_p, deg_in_p, W2, b1.reshape(1, D_H))
    agg2_p = _agg64(h2, src_p, dst_p, z64)
    out = _tc_fin(agg2_p, deg_in_p, b2.reshape(1, D_OUT))
    return out[:N]
